# baseline (speedup 1.0000x reference)
import functools
import math

import jax
import jax.numpy as jnp
from jax import lax
from jax.experimental import pallas as pl
from jax.experimental.pallas import tpu as pltpu
from jax.experimental.pallas import tpu_sc as plsc

_F = 26
_V = 100000
_E = 16
_B = 16384
_NCOL = _F * _E
_SCALE = math.sqrt(_F * _V * _E)

_NC = 2
_NS = 16
_NW = _NC * _NS
_CPW = _NCOL // _NW

_VLO = 50176
_VHI = _V - _VLO

_CB = 2048
_NCHUNK = _B // _CB


def _sc_body(xs_hbm, tab_hbm, out_hbm,
             lo_v, hi_v, or_v, idx0, idx1,
             losem, hisem, isem, osem):
    idx_v = (idx0, idx1)

    c_ax = lax.axis_index("c")
    s_ax = lax.axis_index("s")
    w = s_ax * _NC + c_ax

    def col_fe(j):
        c = w * _CPW + j
        return c, c // _E, lax.rem(c, _E)

    def lo_dma(j):
        _, f, e = col_fe(j)
        return pltpu.make_async_copy(
            tab_hbm.at[f, e, pl.ds(0, _VLO)], lo_v, losem)

    def hi_dma(j):
        _, f, e = col_fe(j)
        return pltpu.make_async_copy(
            tab_hbm.at[f, e, pl.ds(_VLO, _VHI)], hi_v, hisem)

    def idx_dma(f, t, b):
        return pltpu.make_async_copy(
            xs_hbm.at[f, pl.ds(t * _CB, _CB)], idx_v[b], isem)

    def out_dma(c, t):
        return pltpu.make_async_copy(
            or_v.at[pl.ds(t * _CB, _CB)],
            out_hbm.at[c, pl.ds(t * _CB, _CB)], osem)

    lo_dma(0).start()

    for j in range(_CPW):
        c, f, e = col_fe(j)

        idx_dma(f, 0, 0).start()
        lo_dma(j).wait()
        hi_dma(j).start()
        idx_dma(f, 0, 0).wait()

        for t in range(_NCHUNK):
            bsel = t % 2
            if t + 1 < _NCHUNK:
                idx_dma(f, t + 1, 1 - bsel).start()
            ib = idx_v[bsel]

            def glo(i, cr, _t=t, _ib=ib):
                iv = _ib[pl.ds(i * 16, 16)]
                vals = plsc.load_gather(lo_v, [jnp.minimum(iv, _VLO - 1)])
                or_v[pl.ds(_t * _CB + i * 16, 16)] = vals * _SCALE
                return cr
            lax.fori_loop(0, _CB // 16, glo, 0)
            if t + 1 < _NCHUNK:
                idx_dma(f, t + 1, 1 - bsel).wait()

        hi_dma(j).wait()
        if j + 1 < _CPW:
            lo_dma(j + 1).start()

        idx_dma(f, 0, 0).start()
        idx_dma(f, 0, 0).wait()
        for t in range(_NCHUNK):
            bsel = t % 2
            if t + 1 < _NCHUNK:
                idx_dma(f, t + 1, 1 - bsel).start()
            if t >= 2:
                out_dma(c, t - 2).wait()
            ib = idx_v[bsel]

            def ghi(i, cr, _t=t, _ib=ib):
                sl = pl.ds(_t * _CB + i * 16, 16)
                iv = _ib[pl.ds(i * 16, 16)]
                vals = plsc.load_gather(hi_v, [jnp.maximum(iv - _VLO, 0)])
                or_v[sl] = jnp.where(iv >= _VLO, vals * _SCALE, or_v[sl])
                return cr
            lax.fori_loop(0, _CB // 16, ghi, 0)

            out_dma(c, t).start()
            if t + 1 < _NCHUNK:
                idx_dma(f, t + 1, 1 - bsel).wait()

        for t in (_NCHUNK - 2, _NCHUNK - 1):
            out_dma(c, t).wait()


@jax.jit
def _run(xs_t, tab_t):
    mesh = plsc.VectorSubcoreMesh(
        core_axis_name="c", subcore_axis_name="s",
        num_cores=_NC, num_subcores=_NS)
    k = functools.partial(
        pl.kernel,
        out_type=jax.ShapeDtypeStruct((_NCOL, _B), jnp.float32),
        mesh=mesh,
        scratch_types=[
            pltpu.VMEM((_VLO,), jnp.float32),
            pltpu.VMEM((_VHI,), jnp.float32),
            pltpu.VMEM((_B,), jnp.float32),
            pltpu.VMEM((_CB,), jnp.int32),
            pltpu.VMEM((_CB,), jnp.int32),
            pltpu.SemaphoreType.DMA,
            pltpu.SemaphoreType.DMA,
            pltpu.SemaphoreType.DMA,
            pltpu.SemaphoreType.DMA,
        ],
        compiler_params=pltpu.CompilerParams(needs_layout_passes=False),
    )(_sc_body)
    return k(xs_t, tab_t)


def kernel(xs, tables):
    xs_t = jnp.transpose(xs, (1, 0))
    tab_t = jnp.transpose(tables, (0, 2, 1))
    out_t = _run(xs_t, tab_t)
    return jnp.transpose(out_t, (1, 0))


# speedup vs baseline: 2.3299x; 2.3299x over previous
import functools
import math

import jax
import jax.numpy as jnp
from jax import lax
from jax.experimental import pallas as pl
from jax.experimental.pallas import tpu as pltpu
from jax.experimental.pallas import tpu_sc as plsc

_F = 26
_V = 100000
_E = 16
_B = 16384
_NCOL = _F * _E
_SCALE = math.sqrt(_F * _V * _E)

_NC = 2
_NS = 16
_NW = _NC * _NS
_CPW = _NCOL // _NW

_CB = 2048
_NCHUNK = _B // _CB


def _sc_body(xs_hbm, tab_hbm, out_hbm,
             col_v, idx0, idx1, out0, out1,
             csem, isem, osem):
    idx_v = (idx0, idx1)
    out_v = (out0, out1)

    c_ax = lax.axis_index("c")
    s_ax = lax.axis_index("s")
    w = s_ax * _NC + c_ax

    def do_column(j):
        c = w * _CPW + j
        f = c // _E
        e = lax.rem(c, _E)

        col_dma = pltpu.make_async_copy(tab_hbm.at[f, e], col_v, csem)
        col_dma.start()

        i_dma = pltpu.make_async_copy(xs_hbm.at[f, pl.ds(0, _CB)], idx_v[0], isem)
        i_dma.start()
        col_dma.wait()
        i_dma.wait()

        for t in range(_NCHUNK):
            bsel = t % 2
            if t + 1 < _NCHUNK:
                pltpu.make_async_copy(
                    xs_hbm.at[f, pl.ds((t + 1) * _CB, _CB)],
                    idx_v[1 - bsel], isem).start()
            if t >= 2:
                pltpu.make_async_copy(
                    out_v[bsel], out_hbm.at[c, pl.ds((t - 2) * _CB, _CB)],
                    osem).wait()
            ib = idx_v[bsel]
            ob = out_v[bsel]

            @plsc.parallel_loop(0, _CB // 16, unroll=4)
            def gather16(i):
                sl = pl.ds(i * 16, 16)
                vals = plsc.load_gather(col_v, [ib[sl]])
                ob[sl] = vals * _SCALE

            pltpu.make_async_copy(
                ob, out_hbm.at[c, pl.ds(t * _CB, _CB)], osem).start()
            if t + 1 < _NCHUNK:
                pltpu.make_async_copy(
                    xs_hbm.at[f, pl.ds((t + 1) * _CB, _CB)],
                    idx_v[1 - bsel], isem).wait()

        for t in (_NCHUNK - 2, _NCHUNK - 1):
            pltpu.make_async_copy(
                out_v[t % 2], out_hbm.at[c, pl.ds(t * _CB, _CB)], osem).wait()

    for j in range(_CPW):
        do_column(j)


@jax.jit
def _run(xs_t, tab_t):
    mesh = plsc.VectorSubcoreMesh(
        core_axis_name="c", subcore_axis_name="s",
        num_cores=_NC, num_subcores=_NS)
    k = functools.partial(
        pl.kernel,
        out_type=jax.ShapeDtypeStruct((_NCOL, _B), jnp.float32),
        mesh=mesh,
        scratch_types=[
            pltpu.VMEM((_V,), jnp.float32),
            pltpu.VMEM((_CB,), jnp.int32),
            pltpu.VMEM((_CB,), jnp.int32),
            pltpu.VMEM((_CB,), jnp.float32),
            pltpu.VMEM((_CB,), jnp.float32),
            pltpu.SemaphoreType.DMA,
            pltpu.SemaphoreType.DMA,
            pltpu.SemaphoreType.DMA,
        ],
        compiler_params=pltpu.CompilerParams(needs_layout_passes=False),
    )(_sc_body)
    return k(xs_t, tab_t)


def kernel(xs, tables):
    xs_t = jnp.transpose(xs, (1, 0))
    tab_t = jnp.transpose(tables, (0, 2, 1))
    out_t = _run(xs_t, tab_t)
    return jnp.transpose(out_t, (1, 0))
